# baseline (speedup 1.0000x reference)
import jax
import jax.numpy as jnp
from jax.experimental import pallas as pl
from jax.experimental.pallas import tpu as pltpu

_B = 4096
_M = 3
_H = 128
_L0, _L1, _L2 = 20, 20, 50
_D0, _D1, _D2 = 512, 256, 128

_BB = 256
_RB = 512


def _lane_mean(x, n, d):
    acc = x[:, 0:d]
    for t in range(1, n):
        acc = acc + x[:, t * d:(t + 1) * d]
    return acc * (1.0 / n)


def _proj_body(m0, m1, m2, w0, b0, w1, b1, w2, b2, out):
    s0 = jnp.sum(m0[...], axis=1) * (1.0 / _L0)
    s1 = jnp.sum(m1[...], axis=1) * (1.0 / _L1)
    s2 = jnp.sum(m2[...], axis=1) * (1.0 / _L2)
    out[0] = jnp.dot(s0, w0[...], preferred_element_type=jnp.float32) + b0[...]
    out[1] = jnp.dot(s1, w1[...], preferred_element_type=jnp.float32) + b1[...]
    out[2] = jnp.dot(s2, w2[...], preferred_element_type=jnp.float32) + b2[...]


def _head_body(a0, a1, a2, th0, hb0, th1, hb1, wo1, bo1, wo2, bo2, out):
    gx = (a0[...] + a1[...] + a2[...]) * (1.0 / _M)
    g1 = jnp.dot(gx, th0[...], preferred_element_type=jnp.float32) + hb0[...]
    g2 = jnp.dot(jnp.maximum(g1, 0.0), th1[...],
                 preferred_element_type=jnp.float32) + hb1[...]
    wsum = wo1[0:_H] + wo1[_H:2 * _H] + wo1[2 * _H:3 * _H]
    h = jnp.maximum(jnp.dot(g2, wsum, preferred_element_type=jnp.float32)
                    + bo1[...], 0.0)
    out[...] = jnp.dot(h, wo2[...], preferred_element_type=jnp.float32) + bo2[...]


def kernel(mod0, mod1, mod2, Wp0, bp0, Wp1, bp1, Wp2, bp2,
           theta0, hbias0, theta1, hbias1, Wo1, bo1, Wo2, bo2):
    f32 = jnp.float32
    row = lambda v: v.reshape(1, -1)

    def full(shape):
        return pl.BlockSpec(shape, lambda i: (0,) * len(shape))

    feats = pl.pallas_call(
        _proj_body,
        grid=(_B // _BB,),
        in_specs=[
            pl.BlockSpec((_BB, _L0, _D0), lambda i: (i, 0, 0)),
            pl.BlockSpec((_BB, _L1, _D1), lambda i: (i, 0, 0)),
            pl.BlockSpec((_BB, _L2, _D2), lambda i: (i, 0, 0)),
            full((_D0, _H)), full((1, _H)),
            full((_D1, _H)), full((1, _H)),
            full((_D2, _H)), full((1, _H)),
        ],
        out_specs=pl.BlockSpec((_M, _BB, _H), lambda i: (0, i, 0)),
        out_shape=jax.ShapeDtypeStruct((_M, _B, _H), f32),
        compiler_params=pltpu.CompilerParams(
            dimension_semantics=("parallel",)),
    )(mod0, mod1, mod2, Wp0, row(bp0), Wp1, row(bp1), Wp2, row(bp2))

    xcat = feats.reshape(_M * _B, _H)
    a0, a1, a2 = xcat[0::3], xcat[1::3], xcat[2::3]

    out = pl.pallas_call(
        _head_body,
        grid=(_B // _RB,),
        in_specs=[
            pl.BlockSpec((_RB, _H), lambda i: (i, 0)),
            pl.BlockSpec((_RB, _H), lambda i: (i, 0)),
            pl.BlockSpec((_RB, _H), lambda i: (i, 0)),
            full((_H, _H)), full((1, _H)),
            full((_H, _H)), full((1, _H)),
            full((_M * _H, _H)), full((1, _H)),
            full((_H, 64)), full((1, 64)),
        ],
        out_specs=pl.BlockSpec((_RB, 64), lambda i: (i, 0)),
        out_shape=jax.ShapeDtypeStruct((_B, 64), f32),
        compiler_params=pltpu.CompilerParams(
            dimension_semantics=("parallel",)),
    )(a0, a1, a2, theta0, row(hbias0), theta1, row(hbias1),
      Wo1, row(bo1), Wo2, row(bo2))
    return out


# speedup vs baseline: 1.0619x; 1.0619x over previous
import jax
import jax.numpy as jnp
from jax.experimental import pallas as pl
from jax.experimental.pallas import tpu as pltpu

_B = 4096
_M = 3
_H = 128
_L0, _L1, _L2 = 20, 20, 50
_D0, _D1, _D2 = 512, 256, 128

_BB = 128
_RB = 512


def _lane_mean(x, n, d):
    acc = x[:, 0:d]
    for t in range(1, n):
        acc = acc + x[:, t * d:(t + 1) * d]
    return acc * (1.0 / n)


def _proj_body(m0, m1, m2, w0, b0, w1, b1, w2, b2, out):
    s0 = jnp.sum(m0[...], axis=1) * (1.0 / _L0)
    s1 = jnp.sum(m1[...], axis=1) * (1.0 / _L1)
    s2 = jnp.sum(m2[...], axis=1) * (1.0 / _L2)
    out[0] = jnp.dot(s0, w0[...], preferred_element_type=jnp.float32) + b0[...]
    out[1] = jnp.dot(s1, w1[...], preferred_element_type=jnp.float32) + b1[...]
    out[2] = jnp.dot(s2, w2[...], preferred_element_type=jnp.float32) + b2[...]


def _head_body(x, th0, hb0, th1, hb1, wo1, bo1, wo2, bo2, out):
    xv = x[...]
    gx = jnp.sum(xv.reshape(_RB, _M, _H), axis=1) * (1.0 / _M)
    g1 = jnp.dot(gx, th0[...], preferred_element_type=jnp.float32) + hb0[...]
    g2 = jnp.dot(jnp.maximum(g1, 0.0), th1[...],
                 preferred_element_type=jnp.float32) + hb1[...]
    wsum = wo1[0:_H] + wo1[_H:2 * _H] + wo1[2 * _H:3 * _H]
    h = jnp.maximum(jnp.dot(g2, wsum, preferred_element_type=jnp.float32)
                    + bo1[...], 0.0)
    out[...] = jnp.dot(h, wo2[...], preferred_element_type=jnp.float32) + bo2[...]


def kernel(mod0, mod1, mod2, Wp0, bp0, Wp1, bp1, Wp2, bp2,
           theta0, hbias0, theta1, hbias1, Wo1, bo1, Wo2, bo2):
    f32 = jnp.float32
    row = lambda v: v.reshape(1, -1)

    def full(shape):
        return pl.BlockSpec(shape, lambda i: (0,) * len(shape))

    feats = pl.pallas_call(
        _proj_body,
        grid=(_B // _BB,),
        in_specs=[
            pl.BlockSpec((_BB, _L0, _D0), lambda i: (i, 0, 0)),
            pl.BlockSpec((_BB, _L1, _D1), lambda i: (i, 0, 0)),
            pl.BlockSpec((_BB, _L2, _D2), lambda i: (i, 0, 0)),
            full((_D0, _H)), full((1, _H)),
            full((_D1, _H)), full((1, _H)),
            full((_D2, _H)), full((1, _H)),
        ],
        out_specs=pl.BlockSpec((_M, _BB, _H), lambda i: (0, i, 0)),
        out_shape=jax.ShapeDtypeStruct((_M, _B, _H), f32),
        compiler_params=pltpu.CompilerParams(
            dimension_semantics=("parallel",)),
    )(mod0, mod1, mod2, Wp0, row(bp0), Wp1, row(bp1), Wp2, row(bp2))

    xcat = feats.reshape(_M * _B, _H)

    out = pl.pallas_call(
        _head_body,
        grid=(_B // _RB,),
        in_specs=[
            pl.BlockSpec((_M * _RB, _H), lambda i: (i, 0)),
            full((_H, _H)), full((1, _H)),
            full((_H, _H)), full((1, _H)),
            full((_M * _H, _H)), full((1, _H)),
            full((_H, 64)), full((1, 64)),
        ],
        out_specs=pl.BlockSpec((_RB, 64), lambda i: (i, 0)),
        out_shape=jax.ShapeDtypeStruct((_B, 64), f32),
        compiler_params=pltpu.CompilerParams(
            dimension_semantics=("parallel",)),
    )(xcat, theta0, row(hbias0), theta1, row(hbias1),
      Wo1, row(bo1), Wo2, row(bo2))
    return out


# speedup vs baseline: 1.0633x; 1.0014x over previous
import jax
import jax.numpy as jnp
from jax.experimental import pallas as pl
from jax.experimental.pallas import tpu as pltpu

_B = 4096
_M = 3
_H = 128
_L0, _L1, _L2 = 20, 20, 50
_D0, _D1, _D2 = 512, 256, 128

_BB = 128
_RB = 1024


def _lane_mean(x, n, d):
    acc = x[:, 0:d]
    for t in range(1, n):
        acc = acc + x[:, t * d:(t + 1) * d]
    return acc * (1.0 / n)


def _proj_body(m0, m1, m2, w0, b0, w1, b1, w2, b2, out):
    s0 = jnp.sum(m0[...], axis=1) * (1.0 / _L0)
    s1 = jnp.sum(m1[...], axis=1) * (1.0 / _L1)
    s2 = jnp.sum(m2[...], axis=1) * (1.0 / _L2)
    out[0] = jnp.dot(s0, w0[...], preferred_element_type=jnp.float32) + b0[...]
    out[1] = jnp.dot(s1, w1[...], preferred_element_type=jnp.float32) + b1[...]
    out[2] = jnp.dot(s2, w2[...], preferred_element_type=jnp.float32) + b2[...]


def _head_body(x, th0, hb0, th1, hb1, wo1, bo1, wo2, bo2, out):
    xv = x[...]
    gx = jnp.sum(xv.reshape(_RB, _M, _H), axis=1) * (1.0 / _M)
    g1 = jnp.dot(gx, th0[...], preferred_element_type=jnp.float32) + hb0[...]
    g2 = jnp.dot(jnp.maximum(g1, 0.0), th1[...],
                 preferred_element_type=jnp.float32) + hb1[...]
    wsum = wo1[0:_H] + wo1[_H:2 * _H] + wo1[2 * _H:3 * _H]
    h = jnp.maximum(jnp.dot(g2, wsum, preferred_element_type=jnp.float32)
                    + bo1[...], 0.0)
    out[...] = jnp.dot(h, wo2[...], preferred_element_type=jnp.float32) + bo2[...]


def kernel(mod0, mod1, mod2, Wp0, bp0, Wp1, bp1, Wp2, bp2,
           theta0, hbias0, theta1, hbias1, Wo1, bo1, Wo2, bo2):
    f32 = jnp.float32
    row = lambda v: v.reshape(1, -1)

    def full(shape):
        return pl.BlockSpec(shape, lambda i: (0,) * len(shape))

    feats = pl.pallas_call(
        _proj_body,
        grid=(_B // _BB,),
        in_specs=[
            pl.BlockSpec((_BB, _L0, _D0), lambda i: (i, 0, 0)),
            pl.BlockSpec((_BB, _L1, _D1), lambda i: (i, 0, 0)),
            pl.BlockSpec((_BB, _L2, _D2), lambda i: (i, 0, 0)),
            full((_D0, _H)), full((1, _H)),
            full((_D1, _H)), full((1, _H)),
            full((_D2, _H)), full((1, _H)),
        ],
        out_specs=pl.BlockSpec((_M, _BB, _H), lambda i: (0, i, 0)),
        out_shape=jax.ShapeDtypeStruct((_M, _B, _H), f32),
        compiler_params=pltpu.CompilerParams(
            dimension_semantics=("parallel",)),
    )(mod0, mod1, mod2, Wp0, row(bp0), Wp1, row(bp1), Wp2, row(bp2))

    xcat = feats.reshape(_M * _B, _H)

    out = pl.pallas_call(
        _head_body,
        grid=(_B // _RB,),
        in_specs=[
            pl.BlockSpec((_M * _RB, _H), lambda i: (i, 0)),
            full((_H, _H)), full((1, _H)),
            full((_H, _H)), full((1, _H)),
            full((_M * _H, _H)), full((1, _H)),
            full((_H, 64)), full((1, 64)),
        ],
        out_specs=pl.BlockSpec((_RB, 64), lambda i: (i, 0)),
        out_shape=jax.ShapeDtypeStruct((_B, 64), f32),
        compiler_params=pltpu.CompilerParams(
            dimension_semantics=("parallel",)),
    )(xcat, theta0, row(hbias0), theta1, row(hbias1),
      Wo1, row(bo1), Wo2, row(bo2))
    return out


# speedup vs baseline: 1.0681x; 1.0045x over previous
import jax
import jax.numpy as jnp
from jax.experimental import pallas as pl
from jax.experimental.pallas import tpu as pltpu

_B = 4096
_M = 3
_H = 128
_L0, _L1, _L2 = 20, 20, 50
_D0, _D1, _D2 = 512, 256, 128

_BB = 128
_RB = 1024


def _lane_mean(x, n, d):
    acc = x[:, 0:d]
    for t in range(1, n):
        acc = acc + x[:, t * d:(t + 1) * d]
    return acc * (1.0 / n)


def _proj_body(m0, m1, m2, w0, b0, w1, b1, w2, b2, out):
    s0 = jnp.sum(m0[...], axis=1) * (1.0 / _L0)
    s1 = jnp.sum(m1[...], axis=1) * (1.0 / _L1)
    s2 = jnp.sum(m2[...], axis=1) * (1.0 / _L2)
    f0 = jnp.dot(s0, w0[...], preferred_element_type=jnp.float32) + b0[...]
    f1 = jnp.dot(s1, w1[...], preferred_element_type=jnp.float32) + b1[...]
    f2 = jnp.dot(s2, w2[...], preferred_element_type=jnp.float32) + b2[...]
    out[0] = f0.astype(out.dtype)
    out[1] = f1.astype(out.dtype)
    out[2] = f2.astype(out.dtype)


def _head_body(x, th0, hb0, th1, hb1, wo1, bo1, wo2, bo2, out):
    xv = x[...].astype(jnp.float32)
    gx = jnp.sum(xv.reshape(_RB, _M, _H), axis=1) * (1.0 / _M)
    g1 = jnp.dot(gx, th0[...], preferred_element_type=jnp.float32) + hb0[...]
    g2 = jnp.dot(jnp.maximum(g1, 0.0), th1[...],
                 preferred_element_type=jnp.float32) + hb1[...]
    wsum = wo1[0:_H] + wo1[_H:2 * _H] + wo1[2 * _H:3 * _H]
    h = jnp.maximum(jnp.dot(g2, wsum, preferred_element_type=jnp.float32)
                    + bo1[...], 0.0)
    out[...] = jnp.dot(h, wo2[...], preferred_element_type=jnp.float32) + bo2[...]


def kernel(mod0, mod1, mod2, Wp0, bp0, Wp1, bp1, Wp2, bp2,
           theta0, hbias0, theta1, hbias1, Wo1, bo1, Wo2, bo2):
    f32 = jnp.float32
    row = lambda v: v.reshape(1, -1)

    def full(shape):
        return pl.BlockSpec(shape, lambda i: (0,) * len(shape))

    feats = pl.pallas_call(
        _proj_body,
        grid=(_B // _BB,),
        in_specs=[
            pl.BlockSpec((_BB, _L0, _D0), lambda i: (i, 0, 0)),
            pl.BlockSpec((_BB, _L1, _D1), lambda i: (i, 0, 0)),
            pl.BlockSpec((_BB, _L2, _D2), lambda i: (i, 0, 0)),
            full((_D0, _H)), full((1, _H)),
            full((_D1, _H)), full((1, _H)),
            full((_D2, _H)), full((1, _H)),
        ],
        out_specs=pl.BlockSpec((_M, _BB, _H), lambda i: (0, i, 0)),
        out_shape=jax.ShapeDtypeStruct((_M, _B, _H), jnp.bfloat16),
        compiler_params=pltpu.CompilerParams(
            dimension_semantics=("parallel",)),
    )(mod0, mod1, mod2, Wp0, row(bp0), Wp1, row(bp1), Wp2, row(bp2))

    xcat = feats.reshape(_M * _B, _H)

    out = pl.pallas_call(
        _head_body,
        grid=(_B // _RB,),
        in_specs=[
            pl.BlockSpec((_M * _RB, _H), lambda i: (i, 0)),
            full((_H, _H)), full((1, _H)),
            full((_H, _H)), full((1, _H)),
            full((_M * _H, _H)), full((1, _H)),
            full((_H, 64)), full((1, 64)),
        ],
        out_specs=pl.BlockSpec((_RB, 64), lambda i: (i, 0)),
        out_shape=jax.ShapeDtypeStruct((_B, 64), f32),
        compiler_params=pltpu.CompilerParams(
            dimension_semantics=("parallel",)),
    )(xcat, theta0, row(hbias0), theta1, row(hbias1),
      Wo1, row(bo1), Wo2, row(bo2))
    return out


# speedup vs baseline: 1.0723x; 1.0039x over previous
import jax
import jax.numpy as jnp
from jax.experimental import pallas as pl
from jax.experimental.pallas import tpu as pltpu

_B = 4096
_M = 3
_H = 128
_L0, _L1, _L2 = 20, 20, 50
_D0, _D1, _D2 = 512, 256, 128

_BB = 128
_GRID = _B // _BB


def _body(m0, m1, m2, w0, b0, w1, b1, w2, b2,
          th0, hb0, th1, hb1, wo1, bo1, wo2, bo2, out, facc):
    i = pl.program_id(0)
    s0 = jnp.sum(m0[...], axis=1) * (1.0 / _L0)
    s1 = jnp.sum(m1[...], axis=1) * (1.0 / _L1)
    s2 = jnp.sum(m2[...], axis=1) * (1.0 / _L2)
    f0 = jnp.dot(s0, w0[...], preferred_element_type=jnp.float32) + b0[...]
    f1 = jnp.dot(s1, w1[...], preferred_element_type=jnp.float32) + b1[...]
    f2 = jnp.dot(s2, w2[...], preferred_element_type=jnp.float32) + b2[...]
    facc[pl.ds(0 * _B + i * _BB, _BB), :] = f0
    facc[pl.ds(1 * _B + i * _BB, _BB), :] = f1
    facc[pl.ds(2 * _B + i * _BB, _BB), :] = f2

    @pl.when(i == _GRID - 1)
    def _head():
        xv = facc[...]
        gx = jnp.sum(xv.reshape(_B, _M, _H), axis=1) * (1.0 / _M)
        g1 = jnp.dot(gx, th0[...], preferred_element_type=jnp.float32) + hb0[...]
        g2 = jnp.dot(jnp.maximum(g1, 0.0), th1[...],
                     preferred_element_type=jnp.float32) + hb1[...]
        wsum = wo1[0:_H] + wo1[_H:2 * _H] + wo1[2 * _H:3 * _H]
        h = jnp.maximum(jnp.dot(g2, wsum, preferred_element_type=jnp.float32)
                        + bo1[...], 0.0)
        out[...] = jnp.dot(h, wo2[...],
                           preferred_element_type=jnp.float32) + bo2[...]


def kernel(mod0, mod1, mod2, Wp0, bp0, Wp1, bp1, Wp2, bp2,
           theta0, hbias0, theta1, hbias1, Wo1, bo1, Wo2, bo2):
    row = lambda v: v.reshape(1, -1)

    def full(shape):
        return pl.BlockSpec(shape, lambda i: (0,) * len(shape))

    return pl.pallas_call(
        _body,
        grid=(_GRID,),
        in_specs=[
            pl.BlockSpec((_BB, _L0, _D0), lambda i: (i, 0, 0)),
            pl.BlockSpec((_BB, _L1, _D1), lambda i: (i, 0, 0)),
            pl.BlockSpec((_BB, _L2, _D2), lambda i: (i, 0, 0)),
            full((_D0, _H)), full((1, _H)),
            full((_D1, _H)), full((1, _H)),
            full((_D2, _H)), full((1, _H)),
            full((_H, _H)), full((1, _H)),
            full((_H, _H)), full((1, _H)),
            full((_M * _H, _H)), full((1, _H)),
            full((_H, 64)), full((1, 64)),
        ],
        out_specs=pl.BlockSpec((_B, 64), lambda i: (0, 0)),
        out_shape=jax.ShapeDtypeStruct((_B, 64), jnp.float32),
        scratch_shapes=[pltpu.VMEM((_M * _B, _H), jnp.float32)],
        compiler_params=pltpu.CompilerParams(
            dimension_semantics=("arbitrary",)),
    )(mod0, mod1, mod2, Wp0, row(bp0), Wp1, row(bp1), Wp2, row(bp2),
      theta0, row(hbias0), theta1, row(hbias1),
      Wo1, row(bo1), Wo2, row(bo2))


# speedup vs baseline: 1.0949x; 1.0211x over previous
import jax
import jax.numpy as jnp
from jax.experimental import pallas as pl
from jax.experimental.pallas import tpu as pltpu

_B = 4096
_M = 3
_H = 128
_L0, _L1, _L2 = 20, 20, 50
_D0, _D1, _D2 = 512, 256, 128

_BB = 128
_GRID = _B // _BB
_GROWS = 56
_GPAD = 4160


def _body(m0, m1, m2, w0, b0, w1, b1, w2, b2,
          th0, hb0, th1, hb1, wo1, bo1, wo2, bo2, out, gacc):
    i = pl.program_id(0)

    @pl.when(i == 0)
    def _init():
        gacc[...] = jnp.zeros((_GPAD, _H), jnp.float32)

    s0 = jnp.sum(m0[...], axis=1) * (1.0 / _L0)
    s1 = jnp.sum(m1[...], axis=1) * (1.0 / _L1)
    s2 = jnp.sum(m2[...], axis=1) * (1.0 / _L2)
    f0 = jnp.dot(s0, w0[...], preferred_element_type=jnp.float32) + b0[...]
    f1 = jnp.dot(s1, w1[...], preferred_element_type=jnp.float32) + b1[...]
    f2 = jnp.dot(s2, w2[...], preferred_element_type=jnp.float32) + b2[...]

    g_iota = jax.lax.broadcasted_iota(jnp.int32, (_GROWS, _BB), 0)
    r_iota = jax.lax.broadcasted_iota(jnp.int32, (_GROWS, _BB), 1)
    for m, f in ((0, f0), (1, f1), (2, f2)):
        c0 = m * _B + i * _BB
        a0 = ((c0 // 3) // 8) * 8
        lo = 3 * (a0 + g_iota)
        n = c0 + r_iota
        p = jnp.where((lo <= n) & (n < lo + 3), 1.0, 0.0).astype(jnp.float32)
        upd = jnp.dot(p, f, preferred_element_type=jnp.float32)
        gacc[pl.ds(a0, _GROWS), :] = gacc[pl.ds(a0, _GROWS), :] + upd

    @pl.when(i == _GRID - 1)
    def _head():
        gx = gacc[0:_B, :] * (1.0 / _M)
        g1 = jnp.dot(gx, th0[...], preferred_element_type=jnp.float32) + hb0[...]
        g2 = jnp.dot(jnp.maximum(g1, 0.0), th1[...],
                     preferred_element_type=jnp.float32) + hb1[...]
        wsum = wo1[0:_H] + wo1[_H:2 * _H] + wo1[2 * _H:3 * _H]
        h = jnp.maximum(jnp.dot(g2, wsum, preferred_element_type=jnp.float32)
                        + bo1[...], 0.0)
        out[...] = jnp.dot(h, wo2[...],
                           preferred_element_type=jnp.float32) + bo2[...]


def kernel(mod0, mod1, mod2, Wp0, bp0, Wp1, bp1, Wp2, bp2,
           theta0, hbias0, theta1, hbias1, Wo1, bo1, Wo2, bo2):
    row = lambda v: v.reshape(1, -1)

    def full(shape):
        return pl.BlockSpec(shape, lambda i: (0,) * len(shape))

    return pl.pallas_call(
        _body,
        grid=(_GRID,),
        in_specs=[
            pl.BlockSpec((_BB, _L0, _D0), lambda i: (i, 0, 0)),
            pl.BlockSpec((_BB, _L1, _D1), lambda i: (i, 0, 0)),
            pl.BlockSpec((_BB, _L2, _D2), lambda i: (i, 0, 0)),
            full((_D0, _H)), full((1, _H)),
            full((_D1, _H)), full((1, _H)),
            full((_D2, _H)), full((1, _H)),
            full((_H, _H)), full((1, _H)),
            full((_H, _H)), full((1, _H)),
            full((_M * _H, _H)), full((1, _H)),
            full((_H, 64)), full((1, 64)),
        ],
        out_specs=pl.BlockSpec((_B, 64), lambda i: (0, 0)),
        out_shape=jax.ShapeDtypeStruct((_B, 64), jnp.float32),
        scratch_shapes=[pltpu.VMEM((_GPAD, _H), jnp.float32)],
        compiler_params=pltpu.CompilerParams(
            dimension_semantics=("arbitrary",)),
    )(mod0, mod1, mod2, Wp0, row(bp0), Wp1, row(bp1), Wp2, row(bp2),
      theta0, row(hbias0), theta1, row(hbias1),
      Wo1, row(bo1), Wo2, row(bo2))
